# baseline (speedup 1.0000x reference)
import functools

import jax
import jax.numpy as jnp
from jax import lax
from jax.experimental import pallas as pl
from jax.experimental.pallas import tpu as pltpu
from jax.experimental.pallas import tpu_sc as plsc

N = 16777216
NUM_CORES = 2
NUM_SUBCORES = 16
NW = NUM_CORES * NUM_SUBCORES
PER_W = N // NW
CHUNK = 8192
NCHUNK = PER_W // CHUNK
NSLOT = 4
LANES = 16


def _make_sc_add():
    mesh = plsc.VectorSubcoreMesh(core_axis_name="c", subcore_axis_name="s")

    @functools.partial(
        pl.kernel,
        mesh=mesh,
        out_type=jax.ShapeDtypeStruct((N,), jnp.float32),
        scratch_types=[
            pltpu.VMEM((NSLOT, CHUNK), jnp.float32),
            pltpu.VMEM((NSLOT, CHUNK), jnp.float32),
            pltpu.VMEM((NSLOT, CHUNK), jnp.float32),
        ] + [pltpu.SemaphoreType.DMA] * (3 * NSLOT),
    )
    def add_kernel(x_hbm, b_hbm, out_hbm, xv, bv, rv, *sems):
        in_x = sems[0:NSLOT]
        in_b = sems[NSLOT:2 * NSLOT]
        out_s = sems[2 * NSLOT:3 * NSLOT]
        wid = lax.axis_index("s") * NUM_CORES + lax.axis_index("c")
        base = wid * PER_W

        def start_in(s, g):
            off = base + g * CHUNK
            pltpu.async_copy(x_hbm.at[pl.ds(off, CHUNK)], xv.at[s], in_x[s])
            pltpu.async_copy(b_hbm.at[pl.ds(off, CHUNK)], bv.at[s], in_b[s])

        def wait_in(s, g):
            off = base + g * CHUNK
            pltpu.make_async_copy(x_hbm.at[pl.ds(off, CHUNK)], xv.at[s],
                                  in_x[s]).wait()
            pltpu.make_async_copy(b_hbm.at[pl.ds(off, CHUNK)], bv.at[s],
                                  in_b[s]).wait()

        def start_out(s, g):
            off = base + g * CHUNK
            pltpu.async_copy(rv.at[s], out_hbm.at[pl.ds(off, CHUNK)], out_s[s])

        def wait_out(s, g):
            off = base + g * CHUNK
            pltpu.make_async_copy(rv.at[s], out_hbm.at[pl.ds(off, CHUNK)],
                                  out_s[s]).wait()

        def compute(s):
            @plsc.parallel_loop(0, CHUNK, step=LANES, unroll=8)
            def add_body(i):
                sl = pl.ds(i, LANES)
                rv[s, sl] = xv[s, sl] + bv[s, sl]

        for s in range(NSLOT):
            start_in(s, s)
        for s in range(NSLOT):
            wait_in(s, s)
            compute(s)
            start_out(s, s)
            start_in(s, s + NSLOT)

        def group_body(k, carry):
            g = NSLOT * k

            def step(s):
                gg = g + s
                wait_in(s, gg)
                wait_out(s, gg - NSLOT)
                compute(s)
                start_out(s, gg)
                start_in(s, gg + NSLOT)

            for s in range(NSLOT):
                step(s)
            return carry

        lax.fori_loop(1, NCHUNK // NSLOT - 1, group_body, 0)

        for s in range(NSLOT):
            g = NCHUNK - NSLOT + s
            wait_in(s, g)
            wait_out(s, g - NSLOT)
            compute(s)
            start_out(s, g)
        for s in range(NSLOT):
            wait_out(s, NCHUNK - NSLOT + s)

    return add_kernel


_sc_add = _make_sc_add()


def kernel(X, bias, out_idxs):
    del out_idxs
    return _sc_add(X, bias)


# speedup vs baseline: 1.1730x; 1.1730x over previous
import functools

import jax
import jax.numpy as jnp
from jax import lax
from jax.experimental import pallas as pl
from jax.experimental.pallas import tpu as pltpu
from jax.experimental.pallas import tpu_sc as plsc

N = 16777216
N_SC = N // 2
NUM_CORES = 2
NUM_SUBCORES = 16
NW = NUM_CORES * NUM_SUBCORES
PER_W = N_SC // NW
CHUNK = 16384
NCHUNK = PER_W // CHUNK
NSLOT = 2
LANES = 16

TC_BLK = 524288
TC_BLK0 = N_SC // TC_BLK
TC_GRID = (N - N_SC) // TC_BLK


def _make_sc_add():
    mesh = plsc.VectorSubcoreMesh(core_axis_name="c", subcore_axis_name="s")

    @functools.partial(
        pl.kernel,
        mesh=mesh,
        out_type=jax.ShapeDtypeStruct((N,), jnp.float32),
        scratch_types=[
            pltpu.VMEM((NSLOT, CHUNK), jnp.float32),
            pltpu.VMEM((NSLOT, CHUNK), jnp.float32),
            pltpu.VMEM((NSLOT, CHUNK), jnp.float32),
        ] + [pltpu.SemaphoreType.DMA] * (3 * NSLOT),
    )
    def add_kernel(x_hbm, b_hbm, out_hbm, xv, bv, rv, *sems):
        in_x = sems[0:NSLOT]
        in_b = sems[NSLOT:2 * NSLOT]
        out_s = sems[2 * NSLOT:3 * NSLOT]
        wid = lax.axis_index("s") * NUM_CORES + lax.axis_index("c")
        base = wid * PER_W

        def start_in(s, g):
            off = base + g * CHUNK
            pltpu.async_copy(x_hbm.at[pl.ds(off, CHUNK)], xv.at[s], in_x[s])
            pltpu.async_copy(b_hbm.at[pl.ds(off, CHUNK)], bv.at[s], in_b[s])

        def wait_in(s, g):
            off = base + g * CHUNK
            pltpu.make_async_copy(x_hbm.at[pl.ds(off, CHUNK)], xv.at[s],
                                  in_x[s]).wait()
            pltpu.make_async_copy(b_hbm.at[pl.ds(off, CHUNK)], bv.at[s],
                                  in_b[s]).wait()

        def start_out(s, g):
            off = base + g * CHUNK
            pltpu.async_copy(rv.at[s], out_hbm.at[pl.ds(off, CHUNK)], out_s[s])

        def wait_out(s, g):
            off = base + g * CHUNK
            pltpu.make_async_copy(rv.at[s], out_hbm.at[pl.ds(off, CHUNK)],
                                  out_s[s]).wait()

        def compute(s):
            @plsc.parallel_loop(0, CHUNK, step=LANES, unroll=8)
            def add_body(i):
                sl = pl.ds(i, LANES)
                rv[s, sl] = xv[s, sl] + bv[s, sl]

        for s in range(NSLOT):
            start_in(s, s)
        for s in range(NSLOT):
            wait_in(s, s)
            compute(s)
            start_out(s, s)
            start_in(s, s + NSLOT)

        def group_body(k, carry):
            g = NSLOT * k

            def step(s):
                gg = g + s
                wait_in(s, gg)
                wait_out(s, gg - NSLOT)
                compute(s)
                start_out(s, gg)
                start_in(s, gg + NSLOT)

            for s in range(NSLOT):
                step(s)
            return carry

        lax.fori_loop(1, NCHUNK // NSLOT - 1, group_body, 0)

        for s in range(NSLOT):
            g = NCHUNK - NSLOT + s
            wait_in(s, g)
            wait_out(s, g - NSLOT)
            compute(s)
            start_out(s, g)
        for s in range(NSLOT):
            wait_out(s, NCHUNK - NSLOT + s)

    return add_kernel


_sc_add = _make_sc_add()


def _tc_body(x_ref, b_ref, sc_ref, o_ref):
    del sc_ref
    o_ref[...] = x_ref[...] + b_ref[...]


def _tc_add_rest(x, b, sc_out):
    return pl.pallas_call(
        _tc_body,
        grid=(TC_GRID,),
        in_specs=[
            pl.BlockSpec((TC_BLK,), lambda i: (i + TC_BLK0,)),
            pl.BlockSpec((TC_BLK,), lambda i: (i + TC_BLK0,)),
            pl.BlockSpec(memory_space=pl.ANY),
        ],
        out_specs=pl.BlockSpec((TC_BLK,), lambda i: (i + TC_BLK0,)),
        out_shape=jax.ShapeDtypeStruct((N,), jnp.float32),
        input_output_aliases={2: 0},
    )(x, b, sc_out)


def kernel(X, bias, out_idxs):
    del out_idxs
    sc_out = _sc_add(X, bias)
    return _tc_add_rest(X, bias, sc_out)


# speedup vs baseline: 1.2273x; 1.0463x over previous
import functools

import jax
import jax.numpy as jnp
from jax import lax
from jax.experimental import pallas as pl
from jax.experimental.pallas import tpu as pltpu
from jax.experimental.pallas import tpu_sc as plsc

N = 16777216
N_SC = (N * 3) // 8
NUM_CORES = 2
NUM_SUBCORES = 16
NW = NUM_CORES * NUM_SUBCORES
PER_W = N_SC // NW
CHUNK = 16384
NCHUNK = PER_W // CHUNK
NSLOT = 2
LANES = 16

TC_BLK = 524288
TC_BLK0 = N_SC // TC_BLK
TC_GRID = (N - N_SC) // TC_BLK


def _make_sc_add():
    mesh = plsc.VectorSubcoreMesh(core_axis_name="c", subcore_axis_name="s")

    @functools.partial(
        pl.kernel,
        mesh=mesh,
        out_type=jax.ShapeDtypeStruct((N,), jnp.float32),
        scratch_types=[
            pltpu.VMEM((NSLOT, CHUNK), jnp.float32),
            pltpu.VMEM((NSLOT, CHUNK), jnp.float32),
            pltpu.VMEM((NSLOT, CHUNK), jnp.float32),
        ] + [pltpu.SemaphoreType.DMA] * (3 * NSLOT),
    )
    def add_kernel(x_hbm, b_hbm, out_hbm, xv, bv, rv, *sems):
        in_x = sems[0:NSLOT]
        in_b = sems[NSLOT:2 * NSLOT]
        out_s = sems[2 * NSLOT:3 * NSLOT]
        wid = lax.axis_index("s") * NUM_CORES + lax.axis_index("c")
        base = wid * PER_W

        def start_in(s, g):
            off = base + g * CHUNK
            pltpu.async_copy(x_hbm.at[pl.ds(off, CHUNK)], xv.at[s], in_x[s])
            pltpu.async_copy(b_hbm.at[pl.ds(off, CHUNK)], bv.at[s], in_b[s])

        def wait_in(s, g):
            off = base + g * CHUNK
            pltpu.make_async_copy(x_hbm.at[pl.ds(off, CHUNK)], xv.at[s],
                                  in_x[s]).wait()
            pltpu.make_async_copy(b_hbm.at[pl.ds(off, CHUNK)], bv.at[s],
                                  in_b[s]).wait()

        def start_out(s, g):
            off = base + g * CHUNK
            pltpu.async_copy(rv.at[s], out_hbm.at[pl.ds(off, CHUNK)], out_s[s])

        def wait_out(s, g):
            off = base + g * CHUNK
            pltpu.make_async_copy(rv.at[s], out_hbm.at[pl.ds(off, CHUNK)],
                                  out_s[s]).wait()

        def compute(s):
            @plsc.parallel_loop(0, CHUNK, step=LANES, unroll=8)
            def add_body(i):
                sl = pl.ds(i, LANES)
                rv[s, sl] = xv[s, sl] + bv[s, sl]

        for s in range(NSLOT):
            start_in(s, s)
        for s in range(NSLOT):
            wait_in(s, s)
            compute(s)
            start_out(s, s)
            start_in(s, s + NSLOT)

        def group_body(k, carry):
            g = NSLOT * k

            def step(s):
                gg = g + s
                wait_in(s, gg)
                wait_out(s, gg - NSLOT)
                compute(s)
                start_out(s, gg)
                start_in(s, gg + NSLOT)

            for s in range(NSLOT):
                step(s)
            return carry

        lax.fori_loop(1, NCHUNK // NSLOT - 1, group_body, 0)

        for s in range(NSLOT):
            g = NCHUNK - NSLOT + s
            wait_in(s, g)
            wait_out(s, g - NSLOT)
            compute(s)
            start_out(s, g)
        for s in range(NSLOT):
            wait_out(s, NCHUNK - NSLOT + s)

    return add_kernel


_sc_add = _make_sc_add()


def _tc_body(x_ref, b_ref, sc_ref, o_ref):
    del sc_ref
    o_ref[...] = x_ref[...] + b_ref[...]


def _tc_add_rest(x, b, sc_out):
    return pl.pallas_call(
        _tc_body,
        grid=(TC_GRID,),
        in_specs=[
            pl.BlockSpec((TC_BLK,), lambda i: (i + TC_BLK0,)),
            pl.BlockSpec((TC_BLK,), lambda i: (i + TC_BLK0,)),
            pl.BlockSpec(memory_space=pl.ANY),
        ],
        out_specs=pl.BlockSpec((TC_BLK,), lambda i: (i + TC_BLK0,)),
        out_shape=jax.ShapeDtypeStruct((N,), jnp.float32),
        input_output_aliases={2: 0},
    )(x, b, sc_out)


def kernel(X, bias, out_idxs):
    del out_idxs
    sc_out = _sc_add(X, bias)
    return _tc_add_rest(X, bias, sc_out)
